# baseline (speedup 1.0000x reference)
import functools

import jax
import jax.numpy as jnp
from jax import lax
from jax.experimental import pallas as pl
from jax.experimental.pallas import tpu as pltpu
from jax.experimental.pallas import tpu_sc as plsc

_N = 10000
_E = 320000
_D = 128
_G = 16
_L = 64

_NW = 32
_EPW = 10240
_EPAD = _NW * _EPW
_EALLOC = _EPAD + 128
_C = 128
_NCHM = _EPW // _C
_EPS = 2 * _EPW
_KA = 50
_KB = 110
_DCH = 2048
_NDCH = _EPW // _DCH
_NP = 10240
_RPT = _NP // 16

_mesh = plsc.VectorSubcoreMesh(core_axis_name="c", subcore_axis_name="s")


@functools.partial(
    pl.kernel,
    out_type=jax.ShapeDtypeStruct((_NW, _N), jnp.float32),
    mesh=_mesh,
    compiler_params=pltpu.CompilerParams(needs_layout_passes=False),
    scratch_types=[
        pltpu.VMEM((_DCH,), jnp.int32),
        pltpu.VMEM((_DCH,), jnp.float32),
        pltpu.VMEM((_N,), jnp.float32),
    ],
)
def _deg(dst_hbm, ew_hbm, out_hbm, dstb, ewb, degv):
    c = lax.axis_index("c")
    s = lax.axis_index("s")
    w = s * 2 + c

    def zero(i, carry):
        degv[pl.ds(i * 16, 16)] = jnp.zeros((16,), jnp.float32)
        return carry

    lax.fori_loop(0, _N // 16, zero, 0)

    def chunk(i, carry):
        base = w * _EPW + i * _DCH
        pltpu.sync_copy(dst_hbm.at[pl.ds(base, _DCH)], dstb)
        pltpu.sync_copy(ew_hbm.at[pl.ds(base, _DCH)], ewb)

        def e16(j, inner):
            idx = dstb[pl.ds(j * 16, 16)]
            val = ewb[pl.ds(j * 16, 16)]
            plsc.addupdate_scatter(degv, [idx], val)
            return inner

        lax.fori_loop(0, _DCH // 16, e16, 0)
        return carry

    lax.fori_loop(0, _NDCH, chunk, 0)
    pltpu.sync_copy(degv, out_hbm.at[w])


@functools.partial(
    pl.kernel,
    out_type=jax.ShapeDtypeStruct((2, _NP, _D), jnp.float32),
    mesh=_mesh,
    compiler_params=pltpu.CompilerParams(needs_layout_passes=False),
    scratch_types=[
        pltpu.VMEM((_C,), jnp.int32),
        pltpu.VMEM((_C,), jnp.int32),
        pltpu.VMEM((_C,), jnp.int32),
        pltpu.VMEM((_C,), jnp.int32),
        pltpu.VMEM((_C,), jnp.float32),
        pltpu.VMEM((_C,), jnp.float32),
        pltpu.VMEM((_C, _D), jnp.float32),
        pltpu.VMEM((_C, _D), jnp.float32),
        pltpu.VMEM_SHARED((_NP, _D), jnp.float32),
        pltpu.SemaphoreType.DMA,
        pltpu.SemaphoreType.DMA,
        pltpu.SemaphoreType.DMA,
        pltpu.SemaphoreType.DMA,
    ],
)
def _mp(y_hbm, src_hbm, dst_hbm, ew_hbm, zero_hbm, out_hbm,
        srcv0, srcv1, dstv0, dstv1, ewv0, ewv1, rows0, rows1, acc,
        gsem0, gsem1, ssem0, ssem1):
    c = lax.axis_index("c")
    s = lax.axis_index("s")
    start = s * _EPS + c * (_KA * _C)
    npairs = jnp.where(c == 0, _KA // 2, _KB // 2)
    srcv = (srcv0, srcv1)
    dstv = (dstv0, dstv1)
    ewv = (ewv0, ewv1)
    rows = (rows0, rows1)
    gsem = (gsem0, gsem1)
    ssem = (ssem0, ssem1)

    pltpu.sync_copy(zero_hbm.at[pl.ds(s * _RPT, _RPT)], acc.at[pl.ds(s * _RPT, _RPT)])
    plsc.subcore_barrier()

    def stage(g, b):
        base = start + g * _C
        pltpu.sync_copy(src_hbm.at[pl.ds(base, _C)], srcv[b])
        pltpu.sync_copy(dst_hbm.at[pl.ds(base, _C)], dstv[b])
        pltpu.sync_copy(ew_hbm.at[pl.ds(base, _C)], ewv[b])

    def scale(b):
        @plsc.parallel_loop(0, _C, unroll=4)
        def body(j):
            vb = plsc.load_gather(ewv[b], [jnp.full((16,), j, jnp.int32)])
            for f in range(_D // 16):
                sl = pl.ds(f * 16, 16)
                rows[b][j, sl] = rows[b][j, sl] * vb

    stage(0, 0)
    pltpu.async_copy(y_hbm.at[srcv[0]], rows[0], gsem[0])
    pltpu.sync_copy(dst_hbm.at[pl.ds(start, _C)], dstv[1])

    def zrow(j, carry):
        for f in range(_D // 16):
            rows1[j, pl.ds(f * 16, 16)] = jnp.zeros((16,), jnp.float32)
        return carry

    lax.fori_loop(0, _C, zrow, 0)
    pltpu.async_copy(rows[1], acc.at[dstv[1]], ssem[1], add=True)

    def pair(i, carry):
        for b in range(2):
            g = 2 * i + b
            nb = 1 - b
            pltpu.make_async_copy(y_hbm.at[srcv[b]], rows[b], gsem[b]).wait()
            pltpu.make_async_copy(rows[nb], acc.at[dstv[nb]], ssem[nb]).wait()
            stage(g + 1, nb)
            pltpu.async_copy(y_hbm.at[srcv[nb]], rows[nb], gsem[nb])
            scale(b)
            pltpu.async_copy(rows[b], acc.at[dstv[b]], ssem[b], add=True)
        return carry

    lax.fori_loop(0, npairs, pair, 0)
    pltpu.make_async_copy(y_hbm.at[srcv[0]], rows[0], gsem[0]).wait()
    pltpu.make_async_copy(rows[1], acc.at[dstv[1]], ssem[1]).wait()
    plsc.subcore_barrier()
    pltpu.sync_copy(acc.at[pl.ds(s * _RPT, _RPT)],
                    out_hbm.at[c, pl.ds(s * _RPT, _RPT)])


def _dinv_body(degp_ref, dinv_ref):
    deg = jnp.sum(degp_ref[...], axis=0) + 1.0
    dinv_ref[...] = lax.rsqrt(deg)[:, None]


_dinv_call = pl.pallas_call(
    _dinv_body,
    out_shape=jax.ShapeDtypeStruct((_N, 1), jnp.float32),
)


_RB = 1000
_NRB = _N // _RB


def _ymm_body(x_ref, w_ref, dinv_ref, y_ref):
    xw = jnp.dot(x_ref[...], w_ref[...], preferred_element_type=jnp.float32)
    y_ref[...] = xw * dinv_ref[...]


_ymm_call = pl.pallas_call(
    _ymm_body,
    grid=(_NRB,),
    in_specs=[
        pl.BlockSpec((_RB, _D), lambda i: (i, 0)),
        pl.BlockSpec((_D, _D), lambda i: (0, 0)),
        pl.BlockSpec((_RB, 1), lambda i: (i, 0)),
    ],
    out_specs=pl.BlockSpec((_RB, _D), lambda i: (i, 0)),
    out_shape=jax.ShapeDtypeStruct((_N, _D), jnp.float32),
)


def _mid_body(acc_ref, y_ref, dinv_ref, b_ref, w_ref, out_ref):
    dinv = dinv_ref[...]
    h = dinv * (acc_ref[0] + acc_ref[1] + y_ref[...]) + b_ref[...][None, :]
    h = jnp.maximum(h, 0.0)
    out_ref[...] = jnp.dot(h, w_ref[...], preferred_element_type=jnp.float32) * dinv


_mid_call = pl.pallas_call(
    _mid_body,
    grid=(_NRB,),
    in_specs=[
        pl.BlockSpec((2, _RB, _D), lambda i: (0, i, 0)),
        pl.BlockSpec((_RB, _D), lambda i: (i, 0)),
        pl.BlockSpec((_RB, 1), lambda i: (i, 0)),
        pl.BlockSpec((_D,), lambda i: (0,)),
        pl.BlockSpec((_D, _D), lambda i: (0, 0)),
    ],
    out_specs=pl.BlockSpec((_RB, _D), lambda i: (i, 0)),
    out_shape=jax.ShapeDtypeStruct((_N, _D), jnp.float32),
)


def _fin_body(acc_ref, y_ref, dinv_ref, b_ref, batch_ref, wfc_ref, bfc_ref,
              out_ref, sums_ref, counts_ref):
    i = pl.program_id(0)

    @pl.when(i == 0)
    def _init():
        sums_ref[...] = jnp.zeros_like(sums_ref)
        counts_ref[...] = jnp.zeros_like(counts_ref)

    dinv = dinv_ref[...]
    h = dinv * (acc_ref[0] + acc_ref[1] + y_ref[...]) + b_ref[...][None, :]
    h = jnp.maximum(h, 0.0)
    bb = batch_ref[...][:, 0]
    onehot = (lax.broadcasted_iota(jnp.int32, (_G, _RB), 0) == bb[None, :]
              ).astype(jnp.float32)
    sums_ref[...] += jnp.dot(onehot, h, preferred_element_type=jnp.float32)
    counts_ref[...] = counts_ref[...] + jnp.sum(onehot, axis=1)[:, None]

    @pl.when(i == _NRB - 1)
    def _done():
        pooled = sums_ref[...] / jnp.maximum(counts_ref[...], 1.0)
        out_ref[...] = (jnp.dot(pooled, wfc_ref[...],
                                preferred_element_type=jnp.float32)
                        + bfc_ref[...][None, :])


_fin_call = pl.pallas_call(
    _fin_body,
    grid=(_NRB,),
    in_specs=[
        pl.BlockSpec((2, _RB, _D), lambda i: (0, i, 0)),
        pl.BlockSpec((_RB, _D), lambda i: (i, 0)),
        pl.BlockSpec((_RB, 1), lambda i: (i, 0)),
        pl.BlockSpec((_D,), lambda i: (0,)),
        pl.BlockSpec((_RB, 1), lambda i: (i, 0)),
        pl.BlockSpec((_D, _L), lambda i: (0, 0)),
        pl.BlockSpec((_L,), lambda i: (0,)),
    ],
    out_specs=pl.BlockSpec((_G, _L), lambda i: (0, 0)),
    out_shape=jax.ShapeDtypeStruct((_G, _L), jnp.float32),
    scratch_shapes=[
        pltpu.VMEM((_G, _D), jnp.float32),
        pltpu.VMEM((_G, _D), jnp.float32),
    ],
)


def kernel(x, edge_index, edge_attr, batch, W1, b1, W2, b2, Wfc, bfc):
    src = edge_index[0]
    dst = edge_index[1]
    pad = _EALLOC - _E
    srcp = jnp.pad(src, (0, pad))
    dstp = jnp.pad(dst, (0, pad))
    ewp = jnp.pad(edge_attr, (0, pad))
    zeros = jnp.zeros((_NP, _D), jnp.float32)

    degp = _deg(dstp, ewp)
    dinv = _dinv_call(degp)
    y1 = _ymm_call(x, W1, dinv)
    acc1 = _mp(y1, srcp, dstp, ewp, zeros)
    y2 = _mid_call(acc1, y1, dinv, b1, W2)
    acc2 = _mp(y2, srcp, dstp, ewp, zeros)
    z = _fin_call(acc2, y2, dinv, b2, batch.reshape(_N, 1), Wfc, bfc)
    return z


# speedup vs baseline: 1.3276x; 1.3276x over previous
import functools

import jax
import jax.numpy as jnp
from jax import lax
from jax.experimental import pallas as pl
from jax.experimental.pallas import tpu as pltpu
from jax.experimental.pallas import tpu_sc as plsc

_N = 10000
_E = 320000
_D = 128
_G = 16
_L = 64

_NW = 32
_EPW = 10240
_EPAD = _NW * _EPW
_EALLOC = _EPAD + 128
_C = 128
_NCHM = _EPW // _C
_EPS = 2 * _EPW
_KA = 110
_KB = 50
_DCH = 2048
_NDCH = _EPW // _DCH
_NP = 10240
_RPT = _NP // 16

_mesh = plsc.VectorSubcoreMesh(core_axis_name="c", subcore_axis_name="s")


@functools.partial(
    pl.kernel,
    out_type=jax.ShapeDtypeStruct((_NW, _N), jnp.float32),
    mesh=_mesh,
    compiler_params=pltpu.CompilerParams(needs_layout_passes=False),
    scratch_types=[
        pltpu.VMEM((_DCH,), jnp.int32),
        pltpu.VMEM((_DCH,), jnp.float32),
        pltpu.VMEM((_N,), jnp.float32),
    ],
)
def _deg(dst_hbm, ew_hbm, out_hbm, dstb, ewb, degv):
    c = lax.axis_index("c")
    s = lax.axis_index("s")
    w = s * 2 + c

    def zero(i, carry):
        degv[pl.ds(i * 16, 16)] = jnp.zeros((16,), jnp.float32)
        return carry

    lax.fori_loop(0, _N // 16, zero, 0)

    def chunk(i, carry):
        base = w * _EPW + i * _DCH
        pltpu.sync_copy(dst_hbm.at[pl.ds(base, _DCH)], dstb)
        pltpu.sync_copy(ew_hbm.at[pl.ds(base, _DCH)], ewb)

        def e16(j, inner):
            idx = dstb[pl.ds(j * 16, 16)]
            val = ewb[pl.ds(j * 16, 16)]
            plsc.addupdate_scatter(degv, [idx], val)
            return inner

        lax.fori_loop(0, _DCH // 16, e16, 0)
        return carry

    lax.fori_loop(0, _NDCH, chunk, 0)
    pltpu.sync_copy(degv, out_hbm.at[w])


@functools.partial(
    pl.kernel,
    out_type=jax.ShapeDtypeStruct((2, _NP, _D), jnp.float32),
    mesh=_mesh,
    compiler_params=pltpu.CompilerParams(needs_layout_passes=False),
    scratch_types=[
        pltpu.VMEM((_C,), jnp.int32),
        pltpu.VMEM((_C,), jnp.int32),
        pltpu.VMEM((_C,), jnp.int32),
        pltpu.VMEM((_C,), jnp.int32),
        pltpu.VMEM((_C,), jnp.float32),
        pltpu.VMEM((_C,), jnp.float32),
        pltpu.VMEM((_C, _D), jnp.float32),
        pltpu.VMEM((_C, _D), jnp.float32),
        pltpu.VMEM_SHARED((_NP, _D), jnp.float32),
        pltpu.SemaphoreType.DMA,
        pltpu.SemaphoreType.DMA,
        pltpu.SemaphoreType.DMA,
        pltpu.SemaphoreType.DMA,
    ],
)
def _mp(y_hbm, src_hbm, dst_hbm, ew_hbm, zero_hbm, out_hbm,
        srcv0, srcv1, dstv0, dstv1, ewv0, ewv1, rows0, rows1, acc,
        gsem0, gsem1, ssem0, ssem1):
    c = lax.axis_index("c")
    s = lax.axis_index("s")
    start = s * _EPS + c * (_KA * _C)
    npairs = jnp.where(c == 0, _KA // 2, _KB // 2)
    srcv = (srcv0, srcv1)
    dstv = (dstv0, dstv1)
    ewv = (ewv0, ewv1)
    rows = (rows0, rows1)
    gsem = (gsem0, gsem1)
    ssem = (ssem0, ssem1)

    pltpu.sync_copy(zero_hbm.at[pl.ds(s * _RPT, _RPT)], acc.at[pl.ds(s * _RPT, _RPT)])
    plsc.subcore_barrier()

    def stage(g, b):
        base = start + g * _C
        pltpu.sync_copy(src_hbm.at[pl.ds(base, _C)], srcv[b])
        pltpu.sync_copy(dst_hbm.at[pl.ds(base, _C)], dstv[b])
        pltpu.sync_copy(ew_hbm.at[pl.ds(base, _C)], ewv[b])

    def scale(b):
        @plsc.parallel_loop(0, _C, unroll=4)
        def body(j):
            vb = plsc.load_gather(ewv[b], [jnp.full((16,), j, jnp.int32)])
            for f in range(_D // 16):
                sl = pl.ds(f * 16, 16)
                rows[b][j, sl] = rows[b][j, sl] * vb

    stage(0, 0)
    pltpu.async_copy(y_hbm.at[srcv[0]], rows[0], gsem[0])
    pltpu.sync_copy(dst_hbm.at[pl.ds(start, _C)], dstv[1])

    def zrow(j, carry):
        for f in range(_D // 16):
            rows1[j, pl.ds(f * 16, 16)] = jnp.zeros((16,), jnp.float32)
        return carry

    lax.fori_loop(0, _C, zrow, 0)
    pltpu.async_copy(rows[1], acc.at[dstv[1]], ssem[1], add=True)

    def pair(i, carry):
        for b in range(2):
            g = 2 * i + b
            nb = 1 - b
            pltpu.make_async_copy(y_hbm.at[srcv[b]], rows[b], gsem[b]).wait()
            pltpu.make_async_copy(rows[nb], acc.at[dstv[nb]], ssem[nb]).wait()
            stage(g + 1, nb)
            pltpu.async_copy(y_hbm.at[srcv[nb]], rows[nb], gsem[nb])
            scale(b)
            pltpu.async_copy(rows[b], acc.at[dstv[b]], ssem[b], add=True)
        return carry

    lax.fori_loop(0, npairs, pair, 0)
    pltpu.make_async_copy(y_hbm.at[srcv[0]], rows[0], gsem[0]).wait()
    pltpu.make_async_copy(rows[1], acc.at[dstv[1]], ssem[1]).wait()
    plsc.subcore_barrier()
    pltpu.sync_copy(acc.at[pl.ds(s * _RPT, _RPT)],
                    out_hbm.at[c, pl.ds(s * _RPT, _RPT)])


def _dinv_body(degp_ref, dinv_ref):
    deg = jnp.sum(degp_ref[...], axis=0) + 1.0
    dinv_ref[...] = lax.rsqrt(deg)[:, None]


_dinv_call = pl.pallas_call(
    _dinv_body,
    out_shape=jax.ShapeDtypeStruct((_N, 1), jnp.float32),
)


_RB = 1000
_NRB = _N // _RB


def _ymm_body(x_ref, w_ref, dinv_ref, y_ref):
    xw = jnp.dot(x_ref[...], w_ref[...], preferred_element_type=jnp.float32)
    y_ref[...] = xw * dinv_ref[...]


_ymm_call = pl.pallas_call(
    _ymm_body,
    grid=(_NRB,),
    in_specs=[
        pl.BlockSpec((_RB, _D), lambda i: (i, 0)),
        pl.BlockSpec((_D, _D), lambda i: (0, 0)),
        pl.BlockSpec((_RB, 1), lambda i: (i, 0)),
    ],
    out_specs=pl.BlockSpec((_RB, _D), lambda i: (i, 0)),
    out_shape=jax.ShapeDtypeStruct((_N, _D), jnp.float32),
)


def _mid_body(acc_ref, y_ref, dinv_ref, b_ref, w_ref, out_ref):
    dinv = dinv_ref[...]
    h = dinv * (acc_ref[0] + acc_ref[1] + y_ref[...]) + b_ref[...][None, :]
    h = jnp.maximum(h, 0.0)
    out_ref[...] = jnp.dot(h, w_ref[...], preferred_element_type=jnp.float32) * dinv


_mid_call = pl.pallas_call(
    _mid_body,
    grid=(_NRB,),
    in_specs=[
        pl.BlockSpec((2, _RB, _D), lambda i: (0, i, 0)),
        pl.BlockSpec((_RB, _D), lambda i: (i, 0)),
        pl.BlockSpec((_RB, 1), lambda i: (i, 0)),
        pl.BlockSpec((_D,), lambda i: (0,)),
        pl.BlockSpec((_D, _D), lambda i: (0, 0)),
    ],
    out_specs=pl.BlockSpec((_RB, _D), lambda i: (i, 0)),
    out_shape=jax.ShapeDtypeStruct((_N, _D), jnp.float32),
)


def _fin_body(acc_ref, y_ref, dinv_ref, b_ref, batch_ref, wfc_ref, bfc_ref,
              out_ref, sums_ref, counts_ref):
    i = pl.program_id(0)

    @pl.when(i == 0)
    def _init():
        sums_ref[...] = jnp.zeros_like(sums_ref)
        counts_ref[...] = jnp.zeros_like(counts_ref)

    dinv = dinv_ref[...]
    h = dinv * (acc_ref[0] + acc_ref[1] + y_ref[...]) + b_ref[...][None, :]
    h = jnp.maximum(h, 0.0)
    bb = batch_ref[...][:, 0]
    onehot = (lax.broadcasted_iota(jnp.int32, (_G, _RB), 0) == bb[None, :]
              ).astype(jnp.float32)
    sums_ref[...] += jnp.dot(onehot, h, preferred_element_type=jnp.float32)
    counts_ref[...] = counts_ref[...] + jnp.sum(onehot, axis=1)[:, None]

    @pl.when(i == _NRB - 1)
    def _done():
        pooled = sums_ref[...] / jnp.maximum(counts_ref[...], 1.0)
        out_ref[...] = (jnp.dot(pooled, wfc_ref[...],
                                preferred_element_type=jnp.float32)
                        + bfc_ref[...][None, :])


_fin_call = pl.pallas_call(
    _fin_body,
    grid=(_NRB,),
    in_specs=[
        pl.BlockSpec((2, _RB, _D), lambda i: (0, i, 0)),
        pl.BlockSpec((_RB, _D), lambda i: (i, 0)),
        pl.BlockSpec((_RB, 1), lambda i: (i, 0)),
        pl.BlockSpec((_D,), lambda i: (0,)),
        pl.BlockSpec((_RB, 1), lambda i: (i, 0)),
        pl.BlockSpec((_D, _L), lambda i: (0, 0)),
        pl.BlockSpec((_L,), lambda i: (0,)),
    ],
    out_specs=pl.BlockSpec((_G, _L), lambda i: (0, 0)),
    out_shape=jax.ShapeDtypeStruct((_G, _L), jnp.float32),
    scratch_shapes=[
        pltpu.VMEM((_G, _D), jnp.float32),
        pltpu.VMEM((_G, _D), jnp.float32),
    ],
)


def kernel(x, edge_index, edge_attr, batch, W1, b1, W2, b2, Wfc, bfc):
    src = edge_index[0]
    dst = edge_index[1]
    pad = _EALLOC - _E
    srcp = jnp.pad(src, (0, pad))
    dstp = jnp.pad(dst, (0, pad))
    ewp = jnp.pad(edge_attr, (0, pad))
    zeros = jnp.zeros((_NP, _D), jnp.float32)

    degp = _deg(dstp, ewp)
    dinv = _dinv_call(degp)
    y1 = _ymm_call(x, W1, dinv)
    acc1 = _mp(y1, srcp, dstp, ewp, zeros)
    y2 = _mid_call(acc1, y1, dinv, b1, W2)
    acc2 = _mp(y2, srcp, dstp, ewp, zeros)
    z = _fin_call(acc2, y2, dinv, b2, batch.reshape(_N, 1), Wfc, bfc)
    return z


# speedup vs baseline: 1.5902x; 1.1978x over previous
import functools

import numpy as np

import jax
import jax.numpy as jnp
from jax import lax
from jax.experimental import pallas as pl
from jax.experimental.pallas import tpu as pltpu
from jax.experimental.pallas import tpu_sc as plsc

_N = 10000
_E = 320000
_D = 128
_G = 16
_L = 64

_NW = 32
_EPW = 10240
_EPAD = _NW * _EPW
_EALLOC = _EPAD + 128
_C = 128
_NCHM = _EPW // _C
_EPS = 2 * _EPW
_KA = 110
_KB = 50
_DCH = 2048
_NDCH = _EPW // _DCH
_NP = 10000
_RPT = _NP // 16

_mesh = plsc.VectorSubcoreMesh(core_axis_name="c", subcore_axis_name="s")


@functools.partial(
    pl.kernel,
    out_type=jax.ShapeDtypeStruct((_NW, _N), jnp.float32),
    mesh=_mesh,
    compiler_params=pltpu.CompilerParams(needs_layout_passes=False),
    scratch_types=[
        pltpu.VMEM((_DCH,), jnp.int32),
        pltpu.VMEM((_DCH,), jnp.float32),
        pltpu.VMEM((_N,), jnp.float32),
    ],
)
def _deg(dst_hbm, ew_hbm, out_hbm, dstb, ewb, degv):
    c = lax.axis_index("c")
    s = lax.axis_index("s")
    w = s * 2 + c

    def zero(i, carry):
        degv[pl.ds(i * 16, 16)] = jnp.zeros((16,), jnp.float32)
        return carry

    lax.fori_loop(0, _N // 16, zero, 0)

    def chunk(i, carry):
        base = w * _EPW + i * _DCH
        pltpu.sync_copy(dst_hbm.at[pl.ds(base, _DCH)], dstb)
        pltpu.sync_copy(ew_hbm.at[pl.ds(base, _DCH)], ewb)

        def e16(j, inner):
            idx = dstb[pl.ds(j * 16, 16)]
            val = ewb[pl.ds(j * 16, 16)]
            plsc.addupdate_scatter(degv, [idx], val)
            return inner

        lax.fori_loop(0, _DCH // 16, e16, 0)
        return carry

    lax.fori_loop(0, _NDCH, chunk, 0)
    pltpu.sync_copy(degv, out_hbm.at[w])


@functools.partial(
    pl.kernel,
    out_type=jax.ShapeDtypeStruct((2, _NP, _D), jnp.float32),
    mesh=_mesh,
    compiler_params=pltpu.CompilerParams(needs_layout_passes=False,
                                         use_tc_tiling_on_sc=False),
    scratch_types=[
        pltpu.VMEM((_C,), jnp.int32),
        pltpu.VMEM((_C,), jnp.int32),
        pltpu.VMEM((_C,), jnp.int32),
        pltpu.VMEM((_C,), jnp.int32),
        pltpu.VMEM((_C,), jnp.float32),
        pltpu.VMEM((_C,), jnp.float32),
        pltpu.VMEM((_C, _D // 2), jnp.int32),
        pltpu.VMEM((_C, _D // 2), jnp.int32),
        pltpu.VMEM((_C, _D), jnp.float32),
        pltpu.VMEM((_C, _D), jnp.float32),
        pltpu.VMEM_SHARED((_NP, _D), jnp.float32),
        pltpu.SemaphoreType.DMA,
        pltpu.SemaphoreType.DMA,
        pltpu.SemaphoreType.DMA,
        pltpu.SemaphoreType.DMA,
    ],
)
def _mp(y_hbm, src_hbm, dst_hbm, ew_hbm, zero_hbm, out_hbm,
        srcv0, srcv1, dstv0, dstv1, ewv0, ewv1, rowsb0, rowsb1,
        rowsf0, rowsf1, acc, gsem0, gsem1, ssem0, ssem1):
    c = lax.axis_index("c")
    s = lax.axis_index("s")
    start = s * _EPS + c * (_KA * _C)
    npairs = jnp.where(c == 0, _KA // 2, _KB // 2)
    srcv = (srcv0, srcv1)
    dstv = (dstv0, dstv1)
    ewv = (ewv0, ewv1)
    rowsb = (rowsb0, rowsb1)
    rowsf = (rowsf0, rowsf1)
    gsem = (gsem0, gsem1)
    ssem = (ssem0, ssem1)

    pltpu.sync_copy(zero_hbm.at[pl.ds(s * _RPT, _RPT)], acc.at[pl.ds(s * _RPT, _RPT)])
    plsc.subcore_barrier()

    def stage(g, b):
        base = start + g * _C
        pltpu.sync_copy(src_hbm.at[pl.ds(base, _C)], srcv[b])
        pltpu.sync_copy(dst_hbm.at[pl.ds(base, _C)], dstv[b])
        pltpu.sync_copy(ew_hbm.at[pl.ds(base, _C)], ewv[b])

    def scale(b):
        @plsc.parallel_loop(0, _C, unroll=4)
        def body(j):
            vb = plsc.load_gather(ewv[b], [jnp.full((16,), j, jnp.int32)])
            for f in range(_D // 32):
                m32 = rowsb[b][j, pl.ds(f * 16, 16)]
                m = plsc.bitcast(m32, jnp.bfloat16)
                lo, hi = plsc.unpack(m, format=plsc.PackFormat.INTERLEAVED)
                rowsf[b][j, pl.ds(f * 32, 16)] = lo * vb
                rowsf[b][j, pl.ds(f * 32 + 16, 16)] = hi * vb

    stage(0, 0)
    pltpu.async_copy(y_hbm.at[srcv[0]], rowsb[0], gsem[0])
    pltpu.sync_copy(dst_hbm.at[pl.ds(start, _C)], dstv[1])

    def zrow(j, carry):
        for f in range(_D // 16):
            rowsf1[j, pl.ds(f * 16, 16)] = jnp.zeros((16,), jnp.float32)
        return carry

    lax.fori_loop(0, _C, zrow, 0)
    pltpu.async_copy(rowsf[1], acc.at[dstv[1]], ssem[1], add=True)

    def pair(i, carry):
        for b in range(2):
            g = 2 * i + b
            nb = 1 - b
            pltpu.make_async_copy(y_hbm.at[srcv[b]], rowsb[b], gsem[b]).wait()
            pltpu.make_async_copy(rowsf[nb], acc.at[dstv[nb]], ssem[nb]).wait()
            stage(g + 1, nb)
            pltpu.async_copy(y_hbm.at[srcv[nb]], rowsb[nb], gsem[nb])
            scale(b)
            pltpu.async_copy(rowsf[b], acc.at[dstv[b]], ssem[b], add=True)
        return carry

    lax.fori_loop(0, npairs, pair, 0)
    pltpu.make_async_copy(y_hbm.at[srcv[0]], rowsb[0], gsem[0]).wait()
    pltpu.make_async_copy(rowsf[1], acc.at[dstv[1]], ssem[1]).wait()
    plsc.subcore_barrier()
    pltpu.sync_copy(acc.at[pl.ds(s * _RPT, _RPT)],
                    out_hbm.at[c, pl.ds(s * _RPT, _RPT)])


def _dinv_body(degp_ref, dinv_ref):
    deg = jnp.sum(degp_ref[...], axis=0) + 1.0
    dinv_ref[...] = lax.rsqrt(deg)[:, None]


_dinv_call = pl.pallas_call(
    _dinv_body,
    out_shape=jax.ShapeDtypeStruct((_N, 1), jnp.float32),
)


_RB = 1000
_NRB = _N // _RB


def _ymm_body(x_ref, w_ref, wt_ref, dinv_ref, yb_ref, yt_ref):
    dinv = dinv_ref[...]
    xw = jnp.dot(x_ref[...], w_ref[...], preferred_element_type=jnp.float32)
    yb_ref[...] = (xw * dinv).astype(jnp.bfloat16)
    xwt = jnp.dot(x_ref[...], wt_ref[...], preferred_element_type=jnp.float32)
    yt_ref[...] = xwt * dinv


_ymm_call = pl.pallas_call(
    _ymm_body,
    grid=(_NRB,),
    in_specs=[
        pl.BlockSpec((_RB, _D), lambda i: (i, 0)),
        pl.BlockSpec((_D, _D), lambda i: (0, 0)),
        pl.BlockSpec((_D, _D), lambda i: (0, 0)),
        pl.BlockSpec((_RB, 1), lambda i: (i, 0)),
    ],
    out_specs=[
        pl.BlockSpec((_RB, _D), lambda i: (i, 0)),
        pl.BlockSpec((_RB, _D), lambda i: (i, 0)),
    ],
    out_shape=[
        jax.ShapeDtypeStruct((_N, _D), jnp.bfloat16),
        jax.ShapeDtypeStruct((_N, _D), jnp.float32),
    ],
)


def _mid_body(acc_ref, yt_ref, dinv_ref, bt_ref, w2r_ref, w2t_ref,
              yb_ref, yt2_ref):
    dinv = dinv_ref[...]
    h = dinv * (acc_ref[0] + acc_ref[1] + yt_ref[...]) + bt_ref[...][None, :]
    h = jnp.maximum(h, 0.0)
    yb_ref[...] = (jnp.dot(h, w2r_ref[...], preferred_element_type=jnp.float32)
                   * dinv).astype(jnp.bfloat16)
    yt2_ref[...] = (jnp.dot(h, w2t_ref[...], preferred_element_type=jnp.float32)
                    * dinv)


_mid_call = pl.pallas_call(
    _mid_body,
    grid=(_NRB,),
    in_specs=[
        pl.BlockSpec((2, _RB, _D), lambda i: (0, i, 0)),
        pl.BlockSpec((_RB, _D), lambda i: (i, 0)),
        pl.BlockSpec((_RB, 1), lambda i: (i, 0)),
        pl.BlockSpec((_D,), lambda i: (0,)),
        pl.BlockSpec((_D, _D), lambda i: (0, 0)),
        pl.BlockSpec((_D, _D), lambda i: (0, 0)),
    ],
    out_specs=[
        pl.BlockSpec((_RB, _D), lambda i: (i, 0)),
        pl.BlockSpec((_RB, _D), lambda i: (i, 0)),
    ],
    out_shape=[
        jax.ShapeDtypeStruct((_N, _D), jnp.bfloat16),
        jax.ShapeDtypeStruct((_N, _D), jnp.float32),
    ],
)


def _fin_body(acc_ref, yt_ref, dinv_ref, bt_ref, batch_ref, wfc_ref, bfc_ref,
              out_ref, sums_ref, counts_ref):
    i = pl.program_id(0)

    @pl.when(i == 0)
    def _init():
        sums_ref[...] = jnp.zeros_like(sums_ref)
        counts_ref[...] = jnp.zeros_like(counts_ref)

    dinv = dinv_ref[...]
    h = dinv * (acc_ref[0] + acc_ref[1] + yt_ref[...]) + bt_ref[...][None, :]
    h = jnp.maximum(h, 0.0)
    bb = batch_ref[...][:, 0]
    onehot = (lax.broadcasted_iota(jnp.int32, (_G, _RB), 0) == bb[None, :]
              ).astype(jnp.float32)
    sums_ref[...] += jnp.dot(onehot, h, preferred_element_type=jnp.float32)
    counts_ref[...] = counts_ref[...] + jnp.sum(onehot, axis=1)[:, None]

    @pl.when(i == _NRB - 1)
    def _done():
        pooled = sums_ref[...] / jnp.maximum(counts_ref[...], 1.0)
        out_ref[...] = (jnp.dot(pooled, wfc_ref[...],
                                preferred_element_type=jnp.float32)
                        + bfc_ref[...][None, :])


_fin_call = pl.pallas_call(
    _fin_body,
    grid=(_NRB,),
    in_specs=[
        pl.BlockSpec((2, _RB, _D), lambda i: (0, i, 0)),
        pl.BlockSpec((_RB, _D), lambda i: (i, 0)),
        pl.BlockSpec((_RB, 1), lambda i: (i, 0)),
        pl.BlockSpec((_D,), lambda i: (0,)),
        pl.BlockSpec((_RB, 1), lambda i: (i, 0)),
        pl.BlockSpec((_D, _L), lambda i: (0, 0)),
        pl.BlockSpec((_L,), lambda i: (0,)),
    ],
    out_specs=pl.BlockSpec((_G, _L), lambda i: (0, 0)),
    out_shape=jax.ShapeDtypeStruct((_G, _L), jnp.float32),
    scratch_shapes=[
        pltpu.VMEM((_G, _D), jnp.float32),
        pltpu.VMEM((_G, _D), jnp.float32),
    ],
)


def _make_tperm():
    perm = []
    for f in range(_D // 32):
        perm.extend(32 * f + 2 * k for k in range(16))
        perm.extend(32 * f + 2 * k + 1 for k in range(16))
    return tuple(perm)


_TPERM = np.array(_make_tperm(), dtype=np.int32)


def kernel(x, edge_index, edge_attr, batch, W1, b1, W2, b2, Wfc, bfc):
    src = edge_index[0]
    dst = edge_index[1]
    pad = _EALLOC - _E
    srcp = jnp.pad(src, (0, pad))
    dstp = jnp.pad(dst, (0, pad))
    ewp = jnp.pad(edge_attr, (0, pad))
    zeros = jnp.zeros((_NP, _D), jnp.float32)

    W1t = W1[:, _TPERM]
    b1t = b1[_TPERM]
    W2r = W2[_TPERM, :]
    W2t = W2r[:, _TPERM]
    b2t = b2[_TPERM]
    Wfct = Wfc[_TPERM, :]

    degp = _deg(dstp, ewp)
    dinv = _dinv_call(degp)
    y1b, y1t = _ymm_call(x, W1, W1t, dinv)
    y1i = lax.bitcast_convert_type(y1b.reshape(_N, _D // 2, 2), jnp.int32)
    acc1 = _mp(y1i, srcp, dstp, ewp, zeros)
    y2b, y2t = _mid_call(acc1, y1t, dinv, b1t, W2r, W2t)
    y2i = lax.bitcast_convert_type(y2b.reshape(_N, _D // 2, 2), jnp.int32)
    acc2 = _mp(y2i, srcp, dstp, ewp, zeros)
    z = _fin_call(acc2, y2t, dinv, b2t, batch.reshape(_N, 1), Wfct, bfc)
    return z


# speedup vs baseline: 1.9983x; 1.2566x over previous
import functools

import numpy as np

import jax
import jax.numpy as jnp
from jax import lax
from jax.experimental import pallas as pl
from jax.experimental.pallas import tpu as pltpu
from jax.experimental.pallas import tpu_sc as plsc

_N = 10000
_E = 320000
_D = 128
_G = 16
_L = 64

_NW = 32
_EPW = 10240
_EPAD = _NW * _EPW
_EALLOC = _EPAD + 128
_C = 128
_NCHM = _EPW // _C
_EPS = 2 * _EPW
_KA = 110
_KB = 50
_DCH = 2048
_NDCH = _EPW // _DCH
_NP = 10000
_RPT = _NP // 16

_mesh = plsc.VectorSubcoreMesh(core_axis_name="c", subcore_axis_name="s")


@functools.partial(
    pl.kernel,
    out_type=jax.ShapeDtypeStruct((_NW, _N), jnp.float32),
    mesh=_mesh,
    compiler_params=pltpu.CompilerParams(needs_layout_passes=False),
    scratch_types=[
        pltpu.VMEM((_DCH,), jnp.int32),
        pltpu.VMEM((_DCH,), jnp.float32),
        pltpu.VMEM((_N,), jnp.float32),
    ],
)
def _deg(dst_hbm, ew_hbm, out_hbm, dstb, ewb, degv):
    c = lax.axis_index("c")
    s = lax.axis_index("s")
    w = s * 2 + c

    def zero(i, carry):
        degv[pl.ds(i * 16, 16)] = jnp.zeros((16,), jnp.float32)
        return carry

    lax.fori_loop(0, _N // 16, zero, 0)

    def chunk(i, carry):
        base = w * _EPW + i * _DCH
        pltpu.sync_copy(dst_hbm.at[pl.ds(base, _DCH)], dstb)
        pltpu.sync_copy(ew_hbm.at[pl.ds(base, _DCH)], ewb)

        def e16(j, inner):
            idx = dstb[pl.ds(j * 16, 16)]
            val = ewb[pl.ds(j * 16, 16)]
            plsc.addupdate_scatter(degv, [idx], val)
            return inner

        lax.fori_loop(0, _DCH // 16, e16, 0)
        return carry

    lax.fori_loop(0, _NDCH, chunk, 0)
    pltpu.sync_copy(degv, out_hbm.at[w])


@functools.partial(
    pl.kernel,
    out_type=jax.ShapeDtypeStruct((2, _NP, _D), jnp.float32),
    mesh=_mesh,
    compiler_params=pltpu.CompilerParams(needs_layout_passes=False,
                                         use_tc_tiling_on_sc=False),
    scratch_types=[
        pltpu.VMEM((_C,), jnp.int32),
        pltpu.VMEM((_C,), jnp.int32),
        pltpu.VMEM((_C,), jnp.int32),
        pltpu.VMEM((_C,), jnp.int32),
        pltpu.VMEM((_C,), jnp.float32),
        pltpu.VMEM((_C,), jnp.float32),
        pltpu.VMEM((_C, _D // 2), jnp.int32),
        pltpu.VMEM((_C, _D // 2), jnp.int32),
        pltpu.VMEM((_C, _D), jnp.float32),
        pltpu.VMEM((_C, _D), jnp.float32),
        pltpu.VMEM_SHARED((_NP, _D), jnp.float32),
        pltpu.SemaphoreType.DMA,
        pltpu.SemaphoreType.DMA,
        pltpu.SemaphoreType.DMA,
        pltpu.SemaphoreType.DMA,
        pltpu.SemaphoreType.DMA,
        pltpu.SemaphoreType.DMA,
    ],
)
def _mp(y_hbm, src_hbm, dst_hbm, ew_hbm, zero_hbm, out_hbm,
        srcv0, srcv1, dstv0, dstv1, ewv0, ewv1, rowsb0, rowsb1,
        rowsf0, rowsf1, acc, gsem0, gsem1, ssem0, ssem1, stsem0, stsem1):
    c = lax.axis_index("c")
    s = lax.axis_index("s")
    start = s * _EPS + c * (_KA * _C)
    npairs = jnp.where(c == 0, _KA // 2, _KB // 2)
    srcv = (srcv0, srcv1)
    dstv = (dstv0, dstv1)
    ewv = (ewv0, ewv1)
    rowsb = (rowsb0, rowsb1)
    rowsf = (rowsf0, rowsf1)
    gsem = (gsem0, gsem1)
    ssem = (ssem0, ssem1)
    stsem = (stsem0, stsem1)

    pltpu.sync_copy(zero_hbm.at[pl.ds(s * _RPT, _RPT)], acc.at[pl.ds(s * _RPT, _RPT)])
    plsc.subcore_barrier()

    def stage(g, b):
        base = start + g * _C
        pltpu.sync_copy(src_hbm.at[pl.ds(base, _C)], srcv[b])
        pltpu.sync_copy(dst_hbm.at[pl.ds(base, _C)], dstv[b])
        pltpu.sync_copy(ew_hbm.at[pl.ds(base, _C)], ewv[b])

    def stage_fire(g, b):
        base = start + g * _C
        pltpu.async_copy(src_hbm.at[pl.ds(base, _C)], srcv[b], stsem[b])
        pltpu.async_copy(dst_hbm.at[pl.ds(base, _C)], dstv[b], stsem[b])
        pltpu.async_copy(ew_hbm.at[pl.ds(base, _C)], ewv[b], stsem[b])

    def stage_wait(g, b):
        base = start + g * _C
        pltpu.make_async_copy(src_hbm.at[pl.ds(base, _C)], srcv[b], stsem[b]).wait()
        pltpu.make_async_copy(dst_hbm.at[pl.ds(base, _C)], dstv[b], stsem[b]).wait()
        pltpu.make_async_copy(ew_hbm.at[pl.ds(base, _C)], ewv[b], stsem[b]).wait()

    def scale(b):
        @plsc.parallel_loop(0, _C, unroll=4)
        def body(j):
            vb = plsc.load_gather(ewv[b], [jnp.full((16,), j, jnp.int32)])
            for f in range(_D // 32):
                m32 = rowsb[b][j, pl.ds(f * 16, 16)]
                m = plsc.bitcast(m32, jnp.bfloat16)
                lo, hi = plsc.unpack(m, format=plsc.PackFormat.INTERLEAVED)
                rowsf[b][j, pl.ds(f * 32, 16)] = lo * vb
                rowsf[b][j, pl.ds(f * 32 + 16, 16)] = hi * vb

    stage(0, 0)
    pltpu.async_copy(y_hbm.at[srcv[0]], rowsb[0], gsem[0])
    pltpu.sync_copy(dst_hbm.at[pl.ds(start, _C)], dstv[1])

    def zrow(j, carry):
        for f in range(_D // 16):
            rowsf1[j, pl.ds(f * 16, 16)] = jnp.zeros((16,), jnp.float32)
        return carry

    lax.fori_loop(0, _C, zrow, 0)
    pltpu.async_copy(rowsf[1], acc.at[dstv[1]], ssem[1], add=True)

    def pair(i, carry):
        for b in range(2):
            g = 2 * i + b
            nb = 1 - b
            pltpu.make_async_copy(y_hbm.at[srcv[b]], rowsb[b], gsem[b]).wait()
            pltpu.make_async_copy(rowsf[nb], acc.at[dstv[nb]], ssem[nb]).wait()
            stage_fire(g + 1, nb)
            scale(b)
            stage_wait(g + 1, nb)
            pltpu.async_copy(y_hbm.at[srcv[nb]], rowsb[nb], gsem[nb])
            pltpu.async_copy(rowsf[b], acc.at[dstv[b]], ssem[b], add=True)
        return carry

    lax.fori_loop(0, npairs, pair, 0)
    pltpu.make_async_copy(y_hbm.at[srcv[0]], rowsb[0], gsem[0]).wait()
    pltpu.make_async_copy(rowsf[1], acc.at[dstv[1]], ssem[1]).wait()
    plsc.subcore_barrier()
    pltpu.sync_copy(acc.at[pl.ds(s * _RPT, _RPT)],
                    out_hbm.at[c, pl.ds(s * _RPT, _RPT)])


def _dinv_body(degp_ref, dinv_ref):
    deg = jnp.sum(degp_ref[...], axis=0) + 1.0
    dinv_ref[...] = lax.rsqrt(deg)[:, None]


_dinv_call = pl.pallas_call(
    _dinv_body,
    out_shape=jax.ShapeDtypeStruct((_N, 1), jnp.float32),
)


_RB = 1000
_NRB = _N // _RB


def _ymm_body(x_ref, w_ref, wt_ref, dinv_ref, yb_ref, yt_ref):
    dinv = dinv_ref[...]
    xw = jnp.dot(x_ref[...], w_ref[...], preferred_element_type=jnp.float32)
    yb_ref[...] = (xw * dinv).astype(jnp.bfloat16)
    xwt = jnp.dot(x_ref[...], wt_ref[...], preferred_element_type=jnp.float32)
    yt_ref[...] = xwt * dinv


_ymm_call = pl.pallas_call(
    _ymm_body,
    grid=(_NRB,),
    in_specs=[
        pl.BlockSpec((_RB, _D), lambda i: (i, 0)),
        pl.BlockSpec((_D, _D), lambda i: (0, 0)),
        pl.BlockSpec((_D, _D), lambda i: (0, 0)),
        pl.BlockSpec((_RB, 1), lambda i: (i, 0)),
    ],
    out_specs=[
        pl.BlockSpec((_RB, _D), lambda i: (i, 0)),
        pl.BlockSpec((_RB, _D), lambda i: (i, 0)),
    ],
    out_shape=[
        jax.ShapeDtypeStruct((_N, _D), jnp.bfloat16),
        jax.ShapeDtypeStruct((_N, _D), jnp.float32),
    ],
)


def _mid_body(acc_ref, yt_ref, dinv_ref, bt_ref, w2r_ref, w2t_ref,
              yb_ref, yt2_ref):
    dinv = dinv_ref[...]
    h = dinv * (acc_ref[0] + acc_ref[1] + yt_ref[...]) + bt_ref[...][None, :]
    h = jnp.maximum(h, 0.0)
    yb_ref[...] = (jnp.dot(h, w2r_ref[...], preferred_element_type=jnp.float32)
                   * dinv).astype(jnp.bfloat16)
    yt2_ref[...] = (jnp.dot(h, w2t_ref[...], preferred_element_type=jnp.float32)
                    * dinv)


_mid_call = pl.pallas_call(
    _mid_body,
    grid=(_NRB,),
    in_specs=[
        pl.BlockSpec((2, _RB, _D), lambda i: (0, i, 0)),
        pl.BlockSpec((_RB, _D), lambda i: (i, 0)),
        pl.BlockSpec((_RB, 1), lambda i: (i, 0)),
        pl.BlockSpec((_D,), lambda i: (0,)),
        pl.BlockSpec((_D, _D), lambda i: (0, 0)),
        pl.BlockSpec((_D, _D), lambda i: (0, 0)),
    ],
    out_specs=[
        pl.BlockSpec((_RB, _D), lambda i: (i, 0)),
        pl.BlockSpec((_RB, _D), lambda i: (i, 0)),
    ],
    out_shape=[
        jax.ShapeDtypeStruct((_N, _D), jnp.bfloat16),
        jax.ShapeDtypeStruct((_N, _D), jnp.float32),
    ],
)


def _fin_body(acc_ref, yt_ref, dinv_ref, bt_ref, batch_ref, wfc_ref, bfc_ref,
              out_ref, sums_ref, counts_ref):
    i = pl.program_id(0)

    @pl.when(i == 0)
    def _init():
        sums_ref[...] = jnp.zeros_like(sums_ref)
        counts_ref[...] = jnp.zeros_like(counts_ref)

    dinv = dinv_ref[...]
    h = dinv * (acc_ref[0] + acc_ref[1] + yt_ref[...]) + bt_ref[...][None, :]
    h = jnp.maximum(h, 0.0)
    bb = batch_ref[...][:, 0]
    onehot = (lax.broadcasted_iota(jnp.int32, (_G, _RB), 0) == bb[None, :]
              ).astype(jnp.float32)
    sums_ref[...] += jnp.dot(onehot, h, preferred_element_type=jnp.float32)
    counts_ref[...] = counts_ref[...] + jnp.sum(onehot, axis=1)[:, None]

    @pl.when(i == _NRB - 1)
    def _done():
        pooled = sums_ref[...] / jnp.maximum(counts_ref[...], 1.0)
        out_ref[...] = (jnp.dot(pooled, wfc_ref[...],
                                preferred_element_type=jnp.float32)
                        + bfc_ref[...][None, :])


_fin_call = pl.pallas_call(
    _fin_body,
    grid=(_NRB,),
    in_specs=[
        pl.BlockSpec((2, _RB, _D), lambda i: (0, i, 0)),
        pl.BlockSpec((_RB, _D), lambda i: (i, 0)),
        pl.BlockSpec((_RB, 1), lambda i: (i, 0)),
        pl.BlockSpec((_D,), lambda i: (0,)),
        pl.BlockSpec((_RB, 1), lambda i: (i, 0)),
        pl.BlockSpec((_D, _L), lambda i: (0, 0)),
        pl.BlockSpec((_L,), lambda i: (0,)),
    ],
    out_specs=pl.BlockSpec((_G, _L), lambda i: (0, 0)),
    out_shape=jax.ShapeDtypeStruct((_G, _L), jnp.float32),
    scratch_shapes=[
        pltpu.VMEM((_G, _D), jnp.float32),
        pltpu.VMEM((_G, _D), jnp.float32),
    ],
)


def _make_tperm():
    perm = []
    for f in range(_D // 32):
        perm.extend(32 * f + 2 * k for k in range(16))
        perm.extend(32 * f + 2 * k + 1 for k in range(16))
    return tuple(perm)


_TPERM = np.array(_make_tperm(), dtype=np.int32)


def kernel(x, edge_index, edge_attr, batch, W1, b1, W2, b2, Wfc, bfc):
    src = edge_index[0]
    dst = edge_index[1]
    pad = _EALLOC - _E
    srcp = jnp.pad(src, (0, pad))
    dstp = jnp.pad(dst, (0, pad))
    ewp = jnp.pad(edge_attr, (0, pad))
    zeros = jnp.zeros((_NP, _D), jnp.float32)

    W1t = W1[:, _TPERM]
    b1t = b1[_TPERM]
    W2r = W2[_TPERM, :]
    W2t = W2r[:, _TPERM]
    b2t = b2[_TPERM]
    Wfct = Wfc[_TPERM, :]

    degp = _deg(dstp, ewp)
    dinv = _dinv_call(degp)
    y1b, y1t = _ymm_call(x, W1, W1t, dinv)
    y1i = lax.bitcast_convert_type(y1b.reshape(_N, _D // 2, 2), jnp.int32)
    acc1 = _mp(y1i, srcp, dstp, ewp, zeros)
    y2b, y2t = _mid_call(acc1, y1t, dinv, b1t, W2r, W2t)
    y2i = lax.bitcast_convert_type(y2b.reshape(_N, _D // 2, 2), jnp.int32)
    acc2 = _mp(y2i, srcp, dstp, ewp, zeros)
    z = _fin_call(acc2, y2t, dinv, b2t, batch.reshape(_N, 1), Wfct, bfc)
    return z
